# baseline (speedup 1.0000x reference)
import functools

import jax
import jax.numpy as jnp
from jax import lax
from jax.experimental import pallas as pl
from jax.experimental.pallas import tpu as pltpu
from jax.experimental.pallas import tpu_sc as plsc

N_CLASSES = 20
N_COMP = 5
D = 256
N = 131072
TAB = 128

NC, NS, L = 2, 16, 16
NW = NC * NS
CHUNK = N // NW
GROUPS = CHUNK // L

B = 4096
NB = N // B


UNROLL = 16


def _routing_body(lab_hbm, lp_hbm, gumt_hbm, idx_hbm, lab_v, gum_v, idx_v, lp_v):
    wid = lax.axis_index("s") * NC + lax.axis_index("c")
    base = wid * CHUNK
    pltpu.sync_copy(lab_hbm.at[pl.ds(base, CHUNK)], lab_v)
    for j in range(N_COMP):
        pltpu.sync_copy(gumt_hbm.at[pl.ds(j * N + base, CHUNK)],
                        gum_v.at[pl.ds(j * CHUNK, CHUNK)])
    pltpu.sync_copy(lp_hbm, lp_v)

    def block(b, _):
        for u in range(UNROLL):
            off = (b * UNROLL + u) * L
            lab = lab_v[pl.ds(off, L)]
            base5 = lab * N_COMP
            safe5 = jnp.minimum(base5, TAB - N_COMP)
            best = plsc.load_gather(lp_v, [safe5]) + gum_v[pl.ds(off, L)]
            comp = jnp.zeros((L,), jnp.int32)
            for j in range(1, N_COMP):
                s = (plsc.load_gather(lp_v, [safe5 + j])
                     + gum_v[pl.ds(j * CHUNK + off, L)])
                upd = s > best
                comp = jnp.where(upd, j, comp)
                best = jnp.where(upd, s, best)
            idx_v[pl.ds(off, L)] = base5 + comp
        return 0

    lax.fori_loop(0, GROUPS // UNROLL, block, 0)
    pltpu.sync_copy(idx_v, idx_hbm.at[pl.ds(base, CHUNK)])


@functools.cache
def _make_routing():
    return pl.kernel(
        _routing_body,
        out_type=jax.ShapeDtypeStruct((N,), jnp.int32),
        mesh=plsc.VectorSubcoreMesh(core_axis_name="c", subcore_axis_name="s",
                                    num_cores=NC, num_subcores=NS),
        scratch_types=[
            pltpu.VMEM((CHUNK,), jnp.int32),
            pltpu.VMEM((CHUNK * N_COMP,), jnp.float32),
            pltpu.VMEM((CHUNK,), jnp.int32),
            pltpu.VMEM((TAB,), jnp.float32),
        ],
        compiler_params=pltpu.CompilerParams(needs_layout_passes=False),
    )


def _dense_body(idx_ref, eps_ref, tab_ref, out_ref):
    idx = idx_ref[...]
    cls = lax.broadcasted_iota(jnp.int32, (TAB, B), 0)
    onehot_t = (idx == cls).astype(jnp.float32)
    prod = lax.dot_general(onehot_t, tab_ref[...],
                           (((0,), (0,)), ((), ())),
                           preferred_element_type=jnp.float32)
    out_ref[...] = prod[:, :D] + jnp.sqrt(prod[:, D:]) * eps_ref[...]


def _dense(idx3, eps, table):
    return pl.pallas_call(
        _dense_body,
        grid=(NB,),
        in_specs=[
            pl.BlockSpec((None, 1, B), lambda i: (i, 0, 0)),
            pl.BlockSpec((B, D), lambda i: (i, 0)),
            pl.BlockSpec((TAB, 2 * D), lambda i: (0, 0)),
        ],
        out_specs=pl.BlockSpec((B, D), lambda i: (i, 0)),
        out_shape=jax.ShapeDtypeStruct((N, D), jnp.float32),
    )(idx3, eps, table)


def kernel(sample_labels, pi, mu, var, eps, gumbel):
    logpi = jnp.log(pi / jnp.sum(pi, axis=-1, keepdims=True))
    lp_flat = jnp.zeros((TAB,), jnp.float32).at[: N_CLASSES * N_COMP].set(
        logpi.reshape(-1))
    table = jnp.zeros((TAB, 2 * D), jnp.float32)
    table = table.at[: N_CLASSES * N_COMP, :D].set(mu.reshape(-1, D))
    table = table.at[: N_CLASSES * N_COMP, D:].set(var.reshape(-1, D))

    idx = _make_routing()(sample_labels.astype(jnp.int32), lp_flat,
                          gumbel.T.reshape(-1))
    return _dense(idx.reshape(NB, 1, B), eps, table)


# speedup vs baseline: 1.0434x; 1.0434x over previous
import functools

import jax
import jax.numpy as jnp
from jax import lax
from jax.experimental import pallas as pl
from jax.experimental.pallas import tpu as pltpu
from jax.experimental.pallas import tpu_sc as plsc

N_CLASSES = 20
N_COMP = 5
D = 256
N = 131072
TAB = 128

NC, NS, L = 2, 16, 16
NW = NC * NS
CHUNK = N // NW
GROUPS = CHUNK // L

B = 8192
NB = N // B


UNROLL = 16


def _routing_body(lab_hbm, lp_hbm, gumt_hbm, idx_hbm, lab_v, gum_v, idx_v, lp_v):
    wid = lax.axis_index("s") * NC + lax.axis_index("c")
    base = wid * CHUNK
    pltpu.sync_copy(lab_hbm.at[pl.ds(base, CHUNK)], lab_v)
    for j in range(N_COMP):
        pltpu.sync_copy(gumt_hbm.at[pl.ds(j * N + base, CHUNK)],
                        gum_v.at[pl.ds(j * CHUNK, CHUNK)])
    pltpu.sync_copy(lp_hbm, lp_v)

    def block(b, _):
        for u in range(UNROLL):
            off = (b * UNROLL + u) * L
            lab = lab_v[pl.ds(off, L)]
            base5 = lab * N_COMP
            safe5 = jnp.minimum(base5, TAB - N_COMP)
            best = plsc.load_gather(lp_v, [safe5]) + gum_v[pl.ds(off, L)]
            comp = jnp.zeros((L,), jnp.int32)
            for j in range(1, N_COMP):
                s = (plsc.load_gather(lp_v, [safe5 + j])
                     + gum_v[pl.ds(j * CHUNK + off, L)])
                upd = s > best
                comp = jnp.where(upd, j, comp)
                best = jnp.where(upd, s, best)
            idx_v[pl.ds(off, L)] = base5 + comp
        return 0

    lax.fori_loop(0, GROUPS // UNROLL, block, 0)
    pltpu.sync_copy(idx_v, idx_hbm.at[pl.ds(base, CHUNK)])


@functools.cache
def _make_routing():
    return pl.kernel(
        _routing_body,
        out_type=jax.ShapeDtypeStruct((N,), jnp.int32),
        mesh=plsc.VectorSubcoreMesh(core_axis_name="c", subcore_axis_name="s",
                                    num_cores=NC, num_subcores=NS),
        scratch_types=[
            pltpu.VMEM((CHUNK,), jnp.int32),
            pltpu.VMEM((CHUNK * N_COMP,), jnp.float32),
            pltpu.VMEM((CHUNK,), jnp.int32),
            pltpu.VMEM((TAB,), jnp.float32),
        ],
        compiler_params=pltpu.CompilerParams(needs_layout_passes=False),
    )


def _dense_body(idx_ref, eps_ref, tab_ref, out_ref):
    idx = idx_ref[...]
    cls = lax.broadcasted_iota(jnp.int32, (TAB, B), 0)
    onehot_t = (idx == cls).astype(jnp.float32)
    prod = lax.dot_general(onehot_t, tab_ref[...],
                           (((0,), (0,)), ((), ())),
                           preferred_element_type=jnp.float32)
    out_ref[...] = prod[:, :D] + jnp.sqrt(prod[:, D:]) * eps_ref[...]


def _dense(idx3, eps, table):
    return pl.pallas_call(
        _dense_body,
        grid=(NB,),
        in_specs=[
            pl.BlockSpec((None, 1, B), lambda i: (i, 0, 0)),
            pl.BlockSpec((B, D), lambda i: (i, 0)),
            pl.BlockSpec((TAB, 2 * D), lambda i: (0, 0)),
        ],
        out_specs=pl.BlockSpec((B, D), lambda i: (i, 0)),
        out_shape=jax.ShapeDtypeStruct((N, D), jnp.float32),
    )(idx3, eps, table)


def kernel(sample_labels, pi, mu, var, eps, gumbel):
    logpi = jnp.log(pi / jnp.sum(pi, axis=-1, keepdims=True))
    lp_flat = jnp.zeros((TAB,), jnp.float32).at[: N_CLASSES * N_COMP].set(
        logpi.reshape(-1))
    table = jnp.zeros((TAB, 2 * D), jnp.float32)
    table = table.at[: N_CLASSES * N_COMP, :D].set(mu.reshape(-1, D))
    table = table.at[: N_CLASSES * N_COMP, D:].set(var.reshape(-1, D))

    idx = _make_routing()(sample_labels.astype(jnp.int32), lp_flat,
                          gumbel.T.reshape(-1))
    return _dense(idx.reshape(NB, 1, B), eps, table)


# speedup vs baseline: 1.0458x; 1.0023x over previous
import functools

import jax
import jax.numpy as jnp
from jax import lax
from jax.experimental import pallas as pl
from jax.experimental.pallas import tpu as pltpu
from jax.experimental.pallas import tpu_sc as plsc

N_CLASSES = 20
N_COMP = 5
D = 256
N = 131072
TAB = 128

NC, NS, L = 2, 16, 16
NW = NC * NS
CHUNK = N // NW
GROUPS = CHUNK // L

B = 8192
NB = N // B


UNROLL = 16


def _routing_body(lab_hbm, lp_hbm, gumt_hbm, idx_hbm, lab_v, gum_v, idx_v, lp_v):
    wid = lax.axis_index("s") * NC + lax.axis_index("c")
    base = wid * CHUNK
    pltpu.sync_copy(lab_hbm.at[pl.ds(base, CHUNK)], lab_v)
    for j in range(N_COMP):
        pltpu.sync_copy(gumt_hbm.at[pl.ds(j * N + base, CHUNK)],
                        gum_v.at[pl.ds(j * CHUNK, CHUNK)])
    pltpu.sync_copy(lp_hbm, lp_v.at[pl.ds(0, N_CLASSES * N_COMP)])

    def block(b, _):
        for u in range(UNROLL):
            off = (b * UNROLL + u) * L
            lab = lab_v[pl.ds(off, L)]
            base5 = lab * N_COMP
            safe5 = jnp.minimum(base5, TAB - N_COMP)
            best = plsc.load_gather(lp_v, [safe5]) + gum_v[pl.ds(off, L)]
            comp = jnp.zeros((L,), jnp.int32)
            for j in range(1, N_COMP):
                s = (plsc.load_gather(lp_v, [safe5 + j])
                     + gum_v[pl.ds(j * CHUNK + off, L)])
                upd = s > best
                comp = jnp.where(upd, j, comp)
                best = jnp.where(upd, s, best)
            idx_v[pl.ds(off, L)] = base5 + comp
        return 0

    lax.fori_loop(0, GROUPS // UNROLL, block, 0)
    pltpu.sync_copy(idx_v, idx_hbm.at[pl.ds(base, CHUNK)])


@functools.cache
def _make_routing():
    return pl.kernel(
        _routing_body,
        out_type=jax.ShapeDtypeStruct((N,), jnp.int32),
        mesh=plsc.VectorSubcoreMesh(core_axis_name="c", subcore_axis_name="s",
                                    num_cores=NC, num_subcores=NS),
        scratch_types=[
            pltpu.VMEM((CHUNK,), jnp.int32),
            pltpu.VMEM((CHUNK * N_COMP,), jnp.float32),
            pltpu.VMEM((CHUNK,), jnp.int32),
            pltpu.VMEM((TAB,), jnp.float32),
        ],
        compiler_params=pltpu.CompilerParams(needs_layout_passes=False),
    )


def _dense_body(idx_ref, eps_ref, mu_ref, var_ref, out_ref):
    idx = idx_ref[...]
    cls = lax.broadcasted_iota(jnp.int32, (TAB, B), 0)
    onehot_t = (idx == cls).astype(jnp.float32)
    dims = (((0,), (0,)), ((), ()))
    mu_s = lax.dot_general(onehot_t, mu_ref[...], dims,
                           preferred_element_type=jnp.float32)
    var_s = lax.dot_general(onehot_t, var_ref[...], dims,
                            preferred_element_type=jnp.float32)
    out_ref[...] = mu_s + jnp.sqrt(var_s) * eps_ref[...]


def _dense(idx3, eps, mu2, var2):
    return pl.pallas_call(
        _dense_body,
        grid=(NB,),
        in_specs=[
            pl.BlockSpec((None, 1, B), lambda i: (i, 0, 0)),
            pl.BlockSpec((B, D), lambda i: (i, 0)),
            pl.BlockSpec((TAB, D), lambda i: (0, 0)),
            pl.BlockSpec((TAB, D), lambda i: (0, 0)),
        ],
        out_specs=pl.BlockSpec((B, D), lambda i: (i, 0)),
        out_shape=jax.ShapeDtypeStruct((N, D), jnp.float32),
    )(idx3, eps, mu2, var2)


def kernel(sample_labels, pi, mu, var, eps, gumbel):
    logpi = jnp.log(pi / jnp.sum(pi, axis=-1, keepdims=True))
    idx = _make_routing()(sample_labels.astype(jnp.int32),
                          logpi.reshape(-1), gumbel.T.reshape(-1))
    return _dense(idx.reshape(NB, 1, B), eps,
                  mu.reshape(-1, D), var.reshape(-1, D))
